# Initial kernel scaffold; baseline (speedup 1.0000x reference)
import jax
import jax.numpy as jnp
from jax.experimental import pallas as pl


def kernel(x, adj_t, W1, b1, W2, b2, W3, b3):
    raise NotImplementedError("write your pallas kernel here")


# speedup vs baseline: 8.3369x; 8.3369x over previous
import functools

import jax
import jax.numpy as jnp
from jax import lax
from jax.experimental import pallas as pl
from jax.experimental.pallas import tpu as pltpu
from jax.experimental.pallas import tpu_sc as plsc

N = 10000
D = 128
NC = 2
NS = 16
NW = NC * NS
WE = 128
NACC = 10240
PTN = NACC // NS
NPAD_ROWS = NACC - N

_mesh = plsc.VectorSubcoreMesh(core_axis_name="c", subcore_axis_name="s")


def _nwin(E):
    return (E + NW * WE - 1) // (NW * WE)


def _make_deg(nwin):
    @functools.partial(
        pl.kernel,
        out_type=jax.ShapeDtypeStruct((NC, NACC), jnp.float32),
        mesh=_mesh,
        scratch_types=[
            pltpu.VMEM((nwin, WE), jnp.int32),
            pltpu.VMEM((PTN,), jnp.float32),
            pltpu.VMEM((WE,), jnp.float32),
            pltpu.VMEM_SHARED((NACC,), jnp.float32),
        ],
    )
    def deg_kernel(dst_hbm, deg_out, dst_v, zv, ones_v, acc):
        c = lax.axis_index("c")
        s = lax.axis_index("s")
        w = c * NS + s

        def fz(i, _):
            zv[pl.ds(i * 16, 16)] = jnp.zeros((16,), jnp.float32)
            return _
        lax.fori_loop(0, PTN // 16, fz, None)

        def fo(i, _):
            ones_v[pl.ds(i * 16, 16)] = jnp.ones((16,), jnp.float32)
            return _
        lax.fori_loop(0, WE // 16, fo, None)

        pltpu.sync_copy(dst_hbm.at[w], dst_v)
        pltpu.sync_copy(zv, acc.at[pl.ds(s * PTN, PTN)])
        plsc.subcore_barrier()

        def body(j, _):
            pltpu.sync_copy(ones_v, acc.at[dst_v.at[j]], add=True)
            return _
        lax.fori_loop(0, nwin, body, None)

        plsc.subcore_barrier()
        pltpu.sync_copy(acc.at[pl.ds(s * PTN, PTN)],
                        deg_out.at[c, pl.ds(s * PTN, PTN)])

    return deg_kernel


def _make_agg(nwin):
    @functools.partial(
        pl.kernel,
        out_type=jax.ShapeDtypeStruct((NC, NACC, D), jnp.float32),
        mesh=_mesh,
        scratch_types=[
            pltpu.VMEM((nwin, WE), jnp.int32),
            pltpu.VMEM((nwin, WE), jnp.int32),
            pltpu.VMEM((WE, D), jnp.float32),
            pltpu.VMEM((64, D), jnp.float32),
            pltpu.VMEM_SHARED((NACC, D), jnp.float32),
        ],
    )
    def agg_kernel(m_hbm, src_hbm, dst_hbm, g_out, src_v, dst_v, buf, zb, acc):
        c = lax.axis_index("c")
        s = lax.axis_index("s")
        w = c * NS + s

        def fz(i, _):
            zb[i // 8, pl.ds((i % 8) * 16, 16)] = jnp.zeros((16,), jnp.float32)
            return _
        lax.fori_loop(0, 64 * 8, fz, None)

        pltpu.sync_copy(src_hbm.at[w], src_v)
        pltpu.sync_copy(dst_hbm.at[w], dst_v)
        base = s * PTN
        for k in range(PTN // 64):
            pltpu.sync_copy(zb, acc.at[pl.ds(base + k * 64, 64)])
        plsc.subcore_barrier()

        def body(j, _):
            pltpu.sync_copy(m_hbm.at[src_v.at[j]], buf)
            pltpu.sync_copy(buf, acc.at[dst_v.at[j]], add=True)
            return _
        lax.fori_loop(0, nwin, body, None)

        plsc.subcore_barrier()
        pltpu.sync_copy(acc.at[pl.ds(base, PTN)],
                        g_out.at[c, pl.ds(base, PTN)])

    return agg_kernel


BR = 400


def _prep_body(x_ref, w_ref, deg_ref, m_ref, dinv_ref):
    deg = deg_ref[0] + deg_ref[1]
    dv = jnp.where(deg > 0.0, lax.rsqrt(jnp.maximum(deg, 1e-12)), 0.0)
    dinv_ref[...] = dv
    h = jax.lax.dot(x_ref[...], w_ref[...],
                    precision=jax.lax.Precision.HIGHEST)
    m_ref[...] = h * dv


def _prep(x, W1, deg2):
    grid = N // BR
    return pl.pallas_call(
        _prep_body,
        grid=(grid,),
        in_specs=[
            pl.BlockSpec((BR, D), lambda i: (i, 0)),
            pl.BlockSpec((D, D), lambda i: (0, 0)),
            pl.BlockSpec((NC, BR, 1), lambda i: (0, i, 0)),
        ],
        out_specs=[
            pl.BlockSpec((BR, D), lambda i: (i, 0)),
            pl.BlockSpec((BR, 1), lambda i: (i, 0)),
        ],
        out_shape=[
            jax.ShapeDtypeStruct((N, D), jnp.float32),
            jax.ShapeDtypeStruct((N, 1), jnp.float32),
        ],
    )(x, W1, deg2)


def _mid_body(g_ref, dinv_ref, b_ref, w_ref, m_ref):
    dv = dinv_ref[...]
    agg = (g_ref[0] + g_ref[1]) * dv + b_ref[...]
    z = jnp.maximum(agg, 0.0)
    h = jax.lax.dot(z, w_ref[...], precision=jax.lax.Precision.HIGHEST)
    m_ref[...] = h * dv


def _mid(g, dinv, b, W):
    grid = N // BR
    return pl.pallas_call(
        _mid_body,
        grid=(grid,),
        in_specs=[
            pl.BlockSpec((NC, BR, D), lambda i: (0, i, 0)),
            pl.BlockSpec((BR, 1), lambda i: (i, 0)),
            pl.BlockSpec((1, D), lambda i: (0, 0)),
            pl.BlockSpec((D, D), lambda i: (0, 0)),
        ],
        out_specs=pl.BlockSpec((BR, D), lambda i: (i, 0)),
        out_shape=jax.ShapeDtypeStruct((N, D), jnp.float32),
    )(g, dinv, b.reshape(1, D), W)


def _final_body(g_ref, dinv_ref, b_ref, o_ref):
    dv = dinv_ref[...]
    o_ref[...] = (g_ref[0] + g_ref[1]) * dv + b_ref[...]


def _final(g, dinv, b):
    grid = N // BR
    return pl.pallas_call(
        _final_body,
        grid=(grid,),
        in_specs=[
            pl.BlockSpec((NC, BR, D), lambda i: (0, i, 0)),
            pl.BlockSpec((BR, 1), lambda i: (i, 0)),
            pl.BlockSpec((1, D), lambda i: (0, 0)),
        ],
        out_specs=pl.BlockSpec((BR, D), lambda i: (i, 0)),
        out_shape=jax.ShapeDtypeStruct((N, D), jnp.float32),
    )(g, dinv, b.reshape(1, D))


def kernel(x, adj_t, W1, b1, W2, b2, W3, b3):
    adj = adj_t.astype(jnp.int32)
    E = adj.shape[1]
    nwin = _nwin(E)
    epad = NW * WE * nwin
    pad = epad - E
    src = jnp.concatenate([adj[0], jnp.zeros((pad,), jnp.int32)])
    dst = jnp.concatenate(
        [adj[1], N + (jnp.arange(pad, dtype=jnp.int32) % NPAD_ROWS)])
    src_w = src.reshape(NW, nwin, WE)
    dst_w = dst.reshape(NW, nwin, WE)

    deg2 = _make_deg(nwin)(dst_w)
    agg = _make_agg(nwin)
    m1, dinv = _prep(x, W1, deg2.reshape(NC, NACC, 1))
    g1 = agg(m1, src_w, dst_w)
    m2 = _mid(g1, dinv, b1, W2)
    g2 = agg(m2, src_w, dst_w)
    m3 = _mid(g2, dinv, b2, W3)
    g3 = agg(m3, src_w, dst_w)
    return _final(g3, dinv, b3)
